# baseline (speedup 1.0000x reference)
import functools

import jax
import jax.numpy as jnp
from jax import lax
from jax.experimental import pallas as pl
from jax.experimental.pallas import tpu as pltpu
from jax.experimental.pallas import tpu_sc as plsc

NC = 2
NS = 16
NW = NC * NS
CH = 128
ZCH = 40


@functools.lru_cache(maxsize=None)
def _make_segment_sum(E, N, D, NB):
    epw = E // NW
    n_full = epw // CH
    tail = epw - n_full * CH
    assert tail % 8 == 0 and n_full >= 2
    n_rounds = (n_full + NB - 1) // NB
    n_zc = D // ZCH
    zc_per_tile = (n_zc + NS - 1) // NS
    Dp = ((D + 127) // 128) * 128
    n_cz = Dp // 128
    cz_per_tile = (n_cz + NS - 1) // NS

    mesh = plsc.VectorSubcoreMesh(core_axis_name="c", subcore_axis_name="s",
                                  num_cores=NC, num_subcores=NS)

    @functools.partial(
        pl.kernel,
        mesh=mesh,
        out_type=(jax.ShapeDtypeStruct((NC, D, 128), jnp.float32),
                  jax.ShapeDtypeStruct((NC * Dp,), jnp.float32)),
        scratch_types=(
            [pltpu.VMEM((CH, 128), jnp.float32) for _ in range(NB)]
            + [
                pltpu.VMEM((n_full * CH,), jnp.int32),
                pltpu.VMEM((n_full * CH,), jnp.int32),
                pltpu.VMEM((tail,), jnp.int32),
                pltpu.VMEM((tail,), jnp.int32),
                pltpu.VMEM((tail, 128), jnp.float32),
                pltpu.VMEM((CH,), jnp.float32),
                pltpu.VMEM((ZCH, 128), jnp.float32),
                pltpu.VMEM((128,), jnp.float32),
                pltpu.VMEM_SHARED((D, 128), jnp.float32),
                pltpu.VMEM_SHARED((Dp,), jnp.float32),
            ]
            + [pltpu.SemaphoreType.DMA for _ in range(3 * NB + 1)]
        ),
    )
    def seg(table, src, dst, acc_out, cnt_out, *refs):
        rows_v = refs[0:NB]
        (src_all, dst_all, srct_v, dstt_v, rowst_v, ones_v, zf_v, zc_v,
         acc_sh, cnt_sh) = refs[NB:NB + 10]
        sem_g = refs[NB + 10:NB + 10 + NB]
        sem_sa = refs[NB + 10 + NB:NB + 10 + 2 * NB]
        sem_sc = refs[NB + 10 + 2 * NB:NB + 10 + 3 * NB]
        sem_z = refs[NB + 10 + 3 * NB]
        cid = lax.axis_index("c")
        sid = lax.axis_index("s")
        wid = cid * NS + sid

        zero16 = jnp.zeros((16,), jnp.float32)
        one16 = jnp.ones((16,), jnp.float32)

        def zrow(j, carry):
            for c in range(8):
                zf_v[j, pl.ds(c * 16, 16)] = zero16
            return carry

        lax.fori_loop(0, ZCH, zrow, 0)
        for r in range(8):
            zc_v[pl.ds(r * 16, 16)] = zero16
        for r in range(CH // 16):
            ones_v[pl.ds(r * 16, 16)] = one16

        base = wid * epw
        pltpu.sync_copy(src.at[pl.ds(base, n_full * CH)], src_all)
        pltpu.sync_copy(dst.at[pl.ds(base, n_full * CH)], dst_all)

        def start_gather(b, c):
            pltpu.async_copy(table.at[src_all.at[pl.ds(c * CH, CH)]],
                             rows_v[b], sem_g[b])

        def wait_gather(b):
            pltpu.make_async_copy(table.at[src_all.at[pl.ds(0, CH)]],
                                  rows_v[b], sem_g[b]).wait()

        def start_scatter(b, c):
            didx = dst_all.at[pl.ds(c * CH, CH)]
            pltpu.async_copy(rows_v[b], acc_sh.at[didx], sem_sa[b], add=True)
            pltpu.async_copy(ones_v, cnt_sh.at[didx], sem_sc[b], add=True)

        def wait_scatter(b):
            didx = dst_all.at[pl.ds(0, CH)]
            pltpu.make_async_copy(rows_v[b], acc_sh.at[didx],
                                  sem_sa[b]).wait()
            pltpu.make_async_copy(ones_v, cnt_sh.at[didx], sem_sc[b]).wait()

        for b in range(2):
            start_gather(b, b)

        def zbody(j, carry):
            ci = j * NS + sid

            @pl.when(ci < n_zc)
            def _():
                pltpu.sync_copy(zf_v, acc_sh.at[pl.ds(ci * ZCH, ZCH)])

            return carry

        lax.fori_loop(0, zc_per_tile, zbody, 0)

        def czbody(j, carry):
            ci = j * NS + sid

            @pl.when(ci < n_cz)
            def _():
                pltpu.sync_copy(zc_v, cnt_sh.at[pl.ds(ci * 128, 128)])

            return carry

        lax.fori_loop(0, cz_per_tile, czbody, 0)
        plsc.subcore_barrier()

        def ebody(k, carry):
            for b in range(NB):
                c = NB * k + b

                @pl.when(c < n_full)
                def _():
                    wait_gather(b)
                    start_scatter(b, c)

                c2 = c + 2
                b2 = (b + 2) % NB

                @pl.when(c2 < n_full)
                def _():
                    @pl.when(c2 >= NB)
                    def _():
                        wait_scatter(b2)

                    start_gather(b2, c2)

            return carry

        lax.fori_loop(0, n_rounds, ebody, 0)

        for b in range(NB):
            c_last = n_full - NB + b
            if c_last >= 0:
                wait_scatter(c_last % NB)

        if tail:
            offt = base + n_full * CH
            pltpu.sync_copy(src.at[pl.ds(offt, tail)], srct_v)
            pltpu.sync_copy(dst.at[pl.ds(offt, tail)], dstt_v)
            pltpu.async_copy(table.at[srct_v], rowst_v, sem_z).wait()
            pltpu.sync_copy(rowst_v, acc_sh.at[dstt_v], add=True)
            pltpu.sync_copy(ones_v.at[pl.ds(0, tail)], cnt_sh.at[dstt_v],
                            add=True)
        plsc.subcore_barrier()

        def obody(j, carry):
            ci = j * NS + sid

            @pl.when(ci < n_zc)
            def _():
                r0 = ci * ZCH
                pltpu.sync_copy(acc_sh.at[pl.ds(r0, ZCH)],
                                acc_out.at[cid, pl.ds(r0, ZCH)])

            return carry

        lax.fori_loop(0, zc_per_tile, obody, 0)

        @pl.when(sid == 0)
        def _():
            pltpu.sync_copy(cnt_sh, cnt_out.at[pl.ds(cid * Dp, Dp)])

    return seg


def _self_mm(x, W_self, b, R, BR, K):
    def body(x_ref, ws_ref, b_ref, o_ref):
        o_ref[...] = (
            jnp.dot(x_ref[...], ws_ref[...], preferred_element_type=jnp.float32)
            + b_ref[...])

    return pl.pallas_call(
        body,
        grid=(R // BR,),
        in_specs=[
            pl.BlockSpec((BR, 128), lambda i: (i, 0)),
            pl.BlockSpec((128, K), lambda i: (0, 0)),
            pl.BlockSpec((1, K), lambda i: (0, 0)),
        ],
        out_specs=pl.BlockSpec((BR, K), lambda i: (i, 0)),
        out_shape=jax.ShapeDtypeStruct((R, K), jnp.float32),
    )(x, W_self, b.reshape(1, K))


def _combine(pre, acc, cnt, W_neigh, R, BR, K, activation):
    def body(p_ref, a_ref, c_ref, wn_ref, o_ref):
        s = a_ref[0] + a_ref[1]
        c = jnp.maximum(c_ref[0] + c_ref[1], 1.0)
        hn = s / c
        r = p_ref[...] + jnp.dot(hn, wn_ref[...],
                                 preferred_element_type=jnp.float32)
        o_ref[...] = jnp.tanh(r) if activation else r

    return pl.pallas_call(
        body,
        grid=(R // BR,),
        in_specs=[
            pl.BlockSpec((BR, K), lambda i: (i, 0)),
            pl.BlockSpec((NC, BR, 128), lambda i: (0, i, 0)),
            pl.BlockSpec((NC, BR, 1), lambda i: (0, i, 0)),
            pl.BlockSpec((128, K), lambda i: (0, 0)),
        ],
        out_specs=pl.BlockSpec((BR, K), lambda i: (i, 0)),
        out_shape=jax.ShapeDtypeStruct((R, K), jnp.float32),
    )(pre, acc, cnt, W_neigh)


def kernel(x, src0, dst0, src1, dst1, num_dst0, num_dst1,
           W1_neigh, W1_self, b1, W2_neigh, W2_self, b2):
    del num_dst0, num_dst1
    src0 = src0.astype(jnp.int32)
    dst0 = dst0.astype(jnp.int32)
    src1 = src1.astype(jnp.int32)
    dst1 = dst1.astype(jnp.int32)
    acc1, cnt1 = _make_segment_sum(160000, 10000, 5000, 4)(x, src0, dst0)
    pre1 = _self_mm(x, W1_self, b1, 5000, 1000, 128)
    h = _combine(pre1, acc1, cnt1.reshape(NC, 5120)[:, :5000, None], W1_neigh,
                 5000, 1000, 128, True)
    acc2, cnt2 = _make_segment_sum(32000, 5000, 1000, 6)(h, src1, dst1)
    pre2 = _self_mm(h, W2_self, b2, 1000, 1000, 64)
    out = _combine(pre2, acc2, cnt2.reshape(NC, 1024)[:, :1000, None],
                   W2_neigh, 1000, 1000, 64, False)
    return out


# speedup vs baseline: 1.0080x; 1.0080x over previous
import functools

import jax
import jax.numpy as jnp
from jax import lax
from jax.experimental import pallas as pl
from jax.experimental.pallas import tpu as pltpu
from jax.experimental.pallas import tpu_sc as plsc

NC = 2
NS = 16
NW = NC * NS
CH = 128
ZCH = 40


@functools.lru_cache(maxsize=None)
def _make_segment_sum(E, N, D, NB):
    epw = E // NW
    n_full = epw // CH
    tail = epw - n_full * CH
    assert tail % 8 == 0 and n_full >= 2
    n_rounds = (n_full + NB - 1) // NB
    n_zc = D // ZCH
    zc_per_tile = (n_zc + NS - 1) // NS
    Dp = ((D + 127) // 128) * 128
    n_cz = Dp // 128
    cz_per_tile = (n_cz + NS - 1) // NS

    mesh = plsc.VectorSubcoreMesh(core_axis_name="c", subcore_axis_name="s",
                                  num_cores=NC, num_subcores=NS)

    @functools.partial(
        pl.kernel,
        mesh=mesh,
        out_type=(jax.ShapeDtypeStruct((NC, D, 128), jnp.float32),
                  jax.ShapeDtypeStruct((NC * Dp,), jnp.float32)),
        scratch_types=(
            [pltpu.VMEM((CH, 128), jnp.float32) for _ in range(NB)]
            + [
                pltpu.VMEM((n_full * CH,), jnp.int32),
                pltpu.VMEM((n_full * CH,), jnp.int32),
                pltpu.VMEM((tail,), jnp.int32),
                pltpu.VMEM((tail,), jnp.int32),
                pltpu.VMEM((tail, 128), jnp.float32),
                pltpu.VMEM((CH,), jnp.float32),
                pltpu.VMEM((ZCH, 128), jnp.float32),
                pltpu.VMEM((128,), jnp.float32),
                pltpu.VMEM_SHARED((D, 128), jnp.float32),
                pltpu.VMEM_SHARED((Dp,), jnp.float32),
            ]
            + [pltpu.SemaphoreType.DMA for _ in range(3 * NB + 1)]
        ),
    )
    def seg(table, src, dst, acc_out, cnt_out, *refs):
        rows_v = refs[0:NB]
        (src_all, dst_all, srct_v, dstt_v, rowst_v, ones_v, zf_v, zc_v,
         acc_sh, cnt_sh) = refs[NB:NB + 10]
        sem_g = refs[NB + 10:NB + 10 + NB]
        sem_sa = refs[NB + 10 + NB:NB + 10 + 2 * NB]
        sem_sc = refs[NB + 10 + 2 * NB:NB + 10 + 3 * NB]
        sem_z = refs[NB + 10 + 3 * NB]
        cid = lax.axis_index("c")
        sid = lax.axis_index("s")
        wid = cid * NS + sid

        zero16 = jnp.zeros((16,), jnp.float32)
        one16 = jnp.ones((16,), jnp.float32)

        def zrow(j, carry):
            for c in range(8):
                zf_v[j, pl.ds(c * 16, 16)] = zero16
            return carry

        lax.fori_loop(0, ZCH, zrow, 0)
        for r in range(8):
            zc_v[pl.ds(r * 16, 16)] = zero16
        for r in range(CH // 16):
            ones_v[pl.ds(r * 16, 16)] = one16

        base = wid * epw
        pltpu.sync_copy(src.at[pl.ds(base, n_full * CH)], src_all)
        pltpu.sync_copy(dst.at[pl.ds(base, n_full * CH)], dst_all)

        def start_gather(b, c):
            pltpu.async_copy(table.at[src_all.at[pl.ds(c * CH, CH)]],
                             rows_v[b], sem_g[b])

        def wait_gather(b):
            pltpu.make_async_copy(table.at[src_all.at[pl.ds(0, CH)]],
                                  rows_v[b], sem_g[b]).wait()

        def start_scatter(b, c):
            didx = dst_all.at[pl.ds(c * CH, CH)]
            pltpu.async_copy(rows_v[b], acc_sh.at[didx], sem_sa[b], add=True)
            pltpu.async_copy(ones_v, cnt_sh.at[didx], sem_sc[b], add=True)

        def wait_scatter(b):
            didx = dst_all.at[pl.ds(0, CH)]
            pltpu.make_async_copy(rows_v[b], acc_sh.at[didx],
                                  sem_sa[b]).wait()
            pltpu.make_async_copy(ones_v, cnt_sh.at[didx], sem_sc[b]).wait()

        for b in range(2):
            start_gather(b, b)

        def zbody(j, carry):
            ci = j * NS + sid

            @pl.when(ci < n_zc)
            def _():
                pltpu.sync_copy(zf_v, acc_sh.at[pl.ds(ci * ZCH, ZCH)])

            return carry

        lax.fori_loop(0, zc_per_tile, zbody, 0)

        def czbody(j, carry):
            ci = j * NS + sid

            @pl.when(ci < n_cz)
            def _():
                pltpu.sync_copy(zc_v, cnt_sh.at[pl.ds(ci * 128, 128)])

            return carry

        lax.fori_loop(0, cz_per_tile, czbody, 0)
        plsc.subcore_barrier()

        def ebody(k, carry):
            for b in range(NB):
                c = NB * k + b

                @pl.when(c < n_full)
                def _():
                    wait_gather(b)
                    start_scatter(b, c)

                c2 = c + 2
                b2 = (b + 2) % NB

                @pl.when(c2 < n_full)
                def _():
                    @pl.when(c2 >= NB)
                    def _():
                        wait_scatter(b2)

                    start_gather(b2, c2)

            return carry

        lax.fori_loop(0, n_rounds, ebody, 0)

        for b in range(NB):
            c_last = n_full - NB + b
            if c_last >= 0:
                wait_scatter(c_last % NB)

        if tail:
            offt = base + n_full * CH
            pltpu.sync_copy(src.at[pl.ds(offt, tail)], srct_v)
            pltpu.sync_copy(dst.at[pl.ds(offt, tail)], dstt_v)
            pltpu.async_copy(table.at[srct_v], rowst_v, sem_z).wait()
            pltpu.sync_copy(rowst_v, acc_sh.at[dstt_v], add=True)
            pltpu.sync_copy(ones_v.at[pl.ds(0, tail)], cnt_sh.at[dstt_v],
                            add=True)
        plsc.subcore_barrier()

        def obody(j, carry):
            ci = j * NS + sid

            @pl.when(ci < n_zc)
            def _():
                r0 = ci * ZCH
                pltpu.sync_copy(acc_sh.at[pl.ds(r0, ZCH)],
                                acc_out.at[cid, pl.ds(r0, ZCH)])

            return carry

        lax.fori_loop(0, zc_per_tile, obody, 0)

        @pl.when(sid == 0)
        def _():
            pltpu.sync_copy(cnt_sh, cnt_out.at[pl.ds(cid * Dp, Dp)])

    return seg


def _self_mm(x, W_self, b, R, BR, K):
    def body(x_ref, ws_ref, b_ref, o_ref):
        o_ref[...] = (
            jnp.dot(x_ref[...], ws_ref[...], preferred_element_type=jnp.float32)
            + b_ref[...])

    return pl.pallas_call(
        body,
        grid=(R // BR,),
        in_specs=[
            pl.BlockSpec((BR, 128), lambda i: (i, 0)),
            pl.BlockSpec((128, K), lambda i: (0, 0)),
            pl.BlockSpec((1, K), lambda i: (0, 0)),
        ],
        out_specs=pl.BlockSpec((BR, K), lambda i: (i, 0)),
        out_shape=jax.ShapeDtypeStruct((R, K), jnp.float32),
    )(x, W_self, b.reshape(1, K))


def _combine(pre, acc, cnt, W_neigh, R, BR, K, activation):
    def body(p_ref, a_ref, c_ref, wn_ref, o_ref):
        s = a_ref[0] + a_ref[1]
        c = jnp.maximum(c_ref[0] + c_ref[1], 1.0)
        hn = s / c
        r = p_ref[...] + jnp.dot(hn, wn_ref[...],
                                 preferred_element_type=jnp.float32)
        o_ref[...] = jnp.tanh(r) if activation else r

    return pl.pallas_call(
        body,
        grid=(R // BR,),
        in_specs=[
            pl.BlockSpec((BR, K), lambda i: (i, 0)),
            pl.BlockSpec((NC, BR, 128), lambda i: (0, i, 0)),
            pl.BlockSpec((NC, BR, 1), lambda i: (0, i, 0)),
            pl.BlockSpec((128, K), lambda i: (0, 0)),
        ],
        out_specs=pl.BlockSpec((BR, K), lambda i: (i, 0)),
        out_shape=jax.ShapeDtypeStruct((R, K), jnp.float32),
    )(pre, acc, cnt, W_neigh)


def kernel(x, src0, dst0, src1, dst1, num_dst0, num_dst1,
           W1_neigh, W1_self, b1, W2_neigh, W2_self, b2):
    del num_dst0, num_dst1
    src0 = src0.astype(jnp.int32)
    dst0 = dst0.astype(jnp.int32)
    src1 = src1.astype(jnp.int32)
    dst1 = dst1.astype(jnp.int32)
    acc1, cnt1 = _make_segment_sum(160000, 10000, 5000, 4)(x, src0, dst0)
    pre1 = _self_mm(x, W1_self, b1, 5000, 5000, 128)
    h = _combine(pre1, acc1, cnt1.reshape(NC, 5120)[:, :5000, None], W1_neigh,
                 5000, 5000, 128, True)
    acc2, cnt2 = _make_segment_sum(32000, 5000, 1000, 6)(h, src1, dst1)
    pre2 = _self_mm(h, W2_self, b2, 1000, 1000, 64)
    out = _combine(pre2, acc2, cnt2.reshape(NC, 1024)[:, :1000, None],
                   W2_neigh, 1000, 1000, 64, False)
    return out
